# baseline (speedup 1.0000x reference)
import functools

import jax
import jax.numpy as jnp
from jax import lax
from jax.experimental import pallas as pl
from jax.experimental.pallas import tpu as pltpu
from jax.experimental.pallas import tpu_sc as plsc

N_NODES = 100000
IN_DIM = 128
NUM_CLASSES = 10
N_GRAPHS = 512
HP = 16

R = 4000
WIN = 64

NT = 84000
NS = N_NODES - NT
NBLK = NT // R

NW = 32
CH = 125
NCH_S = NS // (NW * CH)
GB = NCH_S
NGRP = 1
C0 = 0

_info = plsc.get_sparse_core_info()
_mesh = plsc.VectorSubcoreMesh(
    core_axis_name="c", subcore_axis_name="s", num_cores=_info.num_cores)


def _tc_body(x_ref, b3_ref, wt_ref, out_ref):
    i = pl.program_id(0)

    @pl.when(i == 0)
    def _():
        out_ref[...] = jnp.zeros_like(out_ref)

    h = jnp.dot(x_ref[...], wt_ref[...], preferred_element_type=jnp.float32)
    lane = jax.lax.broadcasted_iota(jnp.int32, (R, HP), 1)
    h_aug = jnp.where(lane == NUM_CLASSES, 1.0, h)

    bids = b3_ref[0, 0, :]
    h_bf = h_aug.astype(jnp.bfloat16)

    g0 = jnp.minimum((bids[0] // 8) * 8, N_GRAPHS - WIN)
    span_ok = (bids[R - 1] - g0) < WIN

    @pl.when(span_ok)
    def _():
        rel = bids - g0
        seg = jax.lax.broadcasted_iota(jnp.int32, (WIN, R), 0)
        onehot_t = (seg == rel[None, :]).astype(jnp.bfloat16)
        upd = jnp.dot(onehot_t, h_bf, preferred_element_type=jnp.float32)
        out_ref[pl.ds(g0, WIN), :] += upd

    @pl.when(jnp.logical_not(span_ok))
    def _():
        seg = jax.lax.broadcasted_iota(jnp.int32, (N_GRAPHS, R), 0)
        onehot_t = (seg == bids[None, :]).astype(jnp.bfloat16)
        out_ref[...] += jnp.dot(onehot_t, h_bf, preferred_element_type=jnp.float32)


@functools.partial(
    pl.kernel,
    mesh=_mesh,
    out_type=[
        jax.ShapeDtypeStruct((2, N_GRAPHS, IN_DIM), jnp.float32),
        jax.ShapeDtypeStruct((2, N_GRAPHS, IN_DIM), jnp.float32),
    ],
    scratch_types=[
        pltpu.VMEM((GB, CH, IN_DIM), jnp.float32),
        pltpu.VMEM((CH,), jnp.int32),
        pltpu.VMEM((CH,), jnp.int32),
        pltpu.VMEM((CH,), jnp.int32),
        pltpu.VMEM((CH,), jnp.int32),
        pltpu.VMEM((CH,), jnp.int32),
        pltpu.VMEM((CH,), jnp.int32),
        pltpu.VMEM((CH,), jnp.int32),
        pltpu.VMEM((CH,), jnp.int32),
        pltpu.VMEM((CH, IN_DIM), jnp.float32),
        pltpu.VMEM_SHARED((N_GRAPHS, IN_DIM), jnp.float32),
        pltpu.VMEM_SHARED((N_GRAPHS, IN_DIM), jnp.float32),
        pltpu.SemaphoreType.DMA,
        pltpu.SemaphoreType.DMA,
    ],
)
def _sc_segsum(x_hbm, batch2_hbm, rix2_hbm, ones_hbm, zvec_hbm,
               outp_hbm, outc_hbm,
               stage_v, i0, i1, i2, i3, r0, r1, r2, r3, ones_v, acc_sh, cnt_sh, sg, ss):
    cid = lax.axis_index("c")
    sid = lax.axis_index("s")
    wid = cid * 16 + sid
    base = C0 + wid * NCH_S
    idxs = [i0, i1, i2, i3]
    rixs = [r0, r1, r2, r3]

    @pl.when(sid == 0)
    def _():
        pltpu.sync_copy(zvec_hbm, acc_sh)
        pltpu.sync_copy(zvec_hbm, cnt_sh)

    pltpu.sync_copy(ones_hbm, ones_v)
    plsc.subcore_barrier()

    hs = []
    for r in range(GB):
        c = base + r
        hs.append(pltpu.async_copy(batch2_hbm.at[c, 0], idxs[r], sg))
        hs.append(pltpu.async_copy(rix2_hbm.at[c, 0], rixs[r], sg))
    for h in hs:
        h.wait()
    hs = []
    for r in range(GB):
        hs.append(pltpu.async_copy(x_hbm.at[rixs[r]], stage_v.at[r], sg))
    sh = []
    for r in range(GB):
        hs[r].wait()
        sh.append(pltpu.async_copy(stage_v.at[r], acc_sh.at[idxs[r]], ss,
                                   add=True))
        sh.append(pltpu.async_copy(ones_v, cnt_sh.at[idxs[r]], ss,
                                   add=True))
    for h in sh:
        h.wait()

    plsc.subcore_barrier()

    @pl.when(sid == 0)
    def _():
        pltpu.sync_copy(acc_sh, outp_hbm.at[cid])
        pltpu.sync_copy(cnt_sh, outc_hbm.at[cid])


def _combine_body(a_ref, p_ref, c_ref, wt_ref, b_ref, o_ref):
    s = p_ref[0:N_GRAPHS, :] + p_ref[N_GRAPHS:2 * N_GRAPHS, :]
    cnt = (c_ref[0:N_GRAPHS, 0:1] + c_ref[N_GRAPHS:2 * N_GRAPHS, 0:1]
           + a_ref[:, NUM_CLASSES:NUM_CLASSES + 1])
    proj = jnp.dot(s, wt_ref[...], preferred_element_type=jnp.float32)
    o_ref[...] = (a_ref[:, :NUM_CLASSES] + proj[:, :NUM_CLASSES]
                  + cnt * b_ref[...])


def kernel(x, edge_index, batch, W, b):
    del edge_index
    wt_pad = jnp.zeros((IN_DIM, HP), jnp.float32).at[:, :NUM_CLASSES].set(W.T)
    batch3 = batch[:NT].reshape(NBLK, 1, R)

    batch2 = batch[NT:].reshape(NS // CH, 1, CH)
    rix2 = (NT + jnp.arange(NS, dtype=jnp.int32)).reshape(NS // CH, 1, CH)
    ones_rows = jnp.ones((CH, IN_DIM), jnp.float32)
    zvec = jnp.zeros((N_GRAPHS, IN_DIM), jnp.float32)

    partials, cnts = _sc_segsum(x, batch2, rix2, ones_rows, zvec)
    p2 = partials.reshape(2 * N_GRAPHS, IN_DIM)
    c2 = cnts.reshape(2 * N_GRAPHS, IN_DIM)

    acc_tc = pl.pallas_call(
        _tc_body,
        grid=(NBLK,),
        in_specs=[
            pl.BlockSpec((R, IN_DIM), lambda i: (i, 0)),
            pl.BlockSpec((1, 1, R), lambda i: (i, 0, 0)),
            pl.BlockSpec((IN_DIM, HP), lambda i: (0, 0)),
        ],
        out_specs=pl.BlockSpec((N_GRAPHS, HP), lambda i: (0, 0)),
        out_shape=jax.ShapeDtypeStruct((N_GRAPHS, HP), jnp.float32),
        compiler_params=pltpu.CompilerParams(
            dimension_semantics=("arbitrary",),
        ),
    )(x, batch3, wt_pad)

    out = pl.pallas_call(
        _combine_body,
        in_specs=[
            pl.BlockSpec((N_GRAPHS, HP), lambda: (0, 0)),
            pl.BlockSpec((2 * N_GRAPHS, IN_DIM), lambda: (0, 0)),
            pl.BlockSpec((2 * N_GRAPHS, IN_DIM), lambda: (0, 0)),
            pl.BlockSpec((IN_DIM, HP), lambda: (0, 0)),
            pl.BlockSpec((1, NUM_CLASSES), lambda: (0, 0)),
        ],
        out_specs=pl.BlockSpec((N_GRAPHS, NUM_CLASSES), lambda: (0, 0)),
        out_shape=jax.ShapeDtypeStruct((N_GRAPHS, NUM_CLASSES), jnp.float32),
    )(acc_tc, p2, c2, wt_pad, b.reshape(1, NUM_CLASSES))
    return out


# speedup vs baseline: 1.0097x; 1.0097x over previous
import functools

import jax
import jax.numpy as jnp
from jax import lax
from jax.experimental import pallas as pl
from jax.experimental.pallas import tpu as pltpu
from jax.experimental.pallas import tpu_sc as plsc

N_NODES = 100000
IN_DIM = 128
NUM_CLASSES = 10
N_GRAPHS = 512
HP = 16

R = 4000
WIN = 64

NT = 88000
NS = N_NODES - NT
NBLK = NT // R

NW = 32
CH = 125
NCH_S = NS // (NW * CH)
GB = NCH_S
NGRP = 1
C0 = 0

_info = plsc.get_sparse_core_info()
_mesh = plsc.VectorSubcoreMesh(
    core_axis_name="c", subcore_axis_name="s", num_cores=_info.num_cores)


def _tc_body(x_ref, b3_ref, wt_ref, out_ref):
    i = pl.program_id(0)

    @pl.when(i == 0)
    def _():
        out_ref[...] = jnp.zeros_like(out_ref)

    h = jnp.dot(x_ref[...], wt_ref[...], preferred_element_type=jnp.float32)
    lane = jax.lax.broadcasted_iota(jnp.int32, (R, HP), 1)
    h_aug = jnp.where(lane == NUM_CLASSES, 1.0, h)

    bids = b3_ref[0, 0, :]
    h_bf = h_aug.astype(jnp.bfloat16)

    g0 = jnp.minimum((bids[0] // 8) * 8, N_GRAPHS - WIN)
    span_ok = (bids[R - 1] - g0) < WIN

    @pl.when(span_ok)
    def _():
        rel = bids - g0
        seg = jax.lax.broadcasted_iota(jnp.int32, (WIN, R), 0)
        onehot_t = (seg == rel[None, :]).astype(jnp.bfloat16)
        upd = jnp.dot(onehot_t, h_bf, preferred_element_type=jnp.float32)
        out_ref[pl.ds(g0, WIN), :] += upd

    @pl.when(jnp.logical_not(span_ok))
    def _():
        seg = jax.lax.broadcasted_iota(jnp.int32, (N_GRAPHS, R), 0)
        onehot_t = (seg == bids[None, :]).astype(jnp.bfloat16)
        out_ref[...] += jnp.dot(onehot_t, h_bf, preferred_element_type=jnp.float32)


@functools.partial(
    pl.kernel,
    mesh=_mesh,
    out_type=[
        jax.ShapeDtypeStruct((2, N_GRAPHS, IN_DIM), jnp.float32),
        jax.ShapeDtypeStruct((2, N_GRAPHS, IN_DIM), jnp.float32),
    ],
    scratch_types=[
        pltpu.VMEM((GB, CH, IN_DIM), jnp.float32),
        pltpu.VMEM((CH,), jnp.int32),
        pltpu.VMEM((CH,), jnp.int32),
        pltpu.VMEM((CH,), jnp.int32),
        pltpu.VMEM((CH,), jnp.int32),
        pltpu.VMEM((CH,), jnp.int32),
        pltpu.VMEM((CH,), jnp.int32),
        pltpu.VMEM((CH, IN_DIM), jnp.float32),
        pltpu.VMEM_SHARED((N_GRAPHS, IN_DIM), jnp.float32),
        pltpu.VMEM_SHARED((N_GRAPHS, IN_DIM), jnp.float32),
        pltpu.SemaphoreType.DMA,
        pltpu.SemaphoreType.DMA,
    ],
)
def _sc_segsum(x_hbm, batch2_hbm, rix2_hbm, ones_hbm, zvec_hbm,
               outp_hbm, outc_hbm,
               stage_v, i0, i1, i2, r0, r1, r2, ones_v, acc_sh, cnt_sh, sg, ss):
    cid = lax.axis_index("c")
    sid = lax.axis_index("s")
    wid = cid * 16 + sid
    base = C0 + wid * NCH_S
    idxs = [i0, i1, i2]
    rixs = [r0, r1, r2]

    @pl.when(sid == 0)
    def _():
        pltpu.sync_copy(zvec_hbm, acc_sh)
        pltpu.sync_copy(zvec_hbm, cnt_sh)

    pltpu.sync_copy(ones_hbm, ones_v)
    plsc.subcore_barrier()

    hs = []
    for r in range(GB):
        c = base + r
        hs.append(pltpu.async_copy(batch2_hbm.at[c, 0], idxs[r], sg))
        hs.append(pltpu.async_copy(rix2_hbm.at[c, 0], rixs[r], sg))
    for h in hs:
        h.wait()
    hs = []
    for r in range(GB):
        hs.append(pltpu.async_copy(x_hbm.at[rixs[r]], stage_v.at[r], sg))
    sh = []
    for r in range(GB):
        hs[r].wait()
        sh.append(pltpu.async_copy(stage_v.at[r], acc_sh.at[idxs[r]], ss,
                                   add=True))
        sh.append(pltpu.async_copy(ones_v, cnt_sh.at[idxs[r]], ss,
                                   add=True))
    for h in sh:
        h.wait()

    plsc.subcore_barrier()

    @pl.when(sid == 0)
    def _():
        pltpu.sync_copy(acc_sh, outp_hbm.at[cid])
        pltpu.sync_copy(cnt_sh, outc_hbm.at[cid])


def _combine_body(a_ref, p_ref, c_ref, wt_ref, b_ref, o_ref):
    s = p_ref[0:N_GRAPHS, :] + p_ref[N_GRAPHS:2 * N_GRAPHS, :]
    cnt = (c_ref[0:N_GRAPHS, 0:1] + c_ref[N_GRAPHS:2 * N_GRAPHS, 0:1]
           + a_ref[:, NUM_CLASSES:NUM_CLASSES + 1])
    proj = jnp.dot(s, wt_ref[...], preferred_element_type=jnp.float32)
    o_ref[...] = (a_ref[:, :NUM_CLASSES] + proj[:, :NUM_CLASSES]
                  + cnt * b_ref[...])


def kernel(x, edge_index, batch, W, b):
    del edge_index
    wt_pad = jnp.zeros((IN_DIM, HP), jnp.float32).at[:, :NUM_CLASSES].set(W.T)
    batch3 = batch[:NT].reshape(NBLK, 1, R)

    batch2 = batch[NT:].reshape(NS // CH, 1, CH)
    rix2 = (NT + jnp.arange(NS, dtype=jnp.int32)).reshape(NS // CH, 1, CH)
    ones_rows = jnp.ones((CH, IN_DIM), jnp.float32)
    zvec = jnp.zeros((N_GRAPHS, IN_DIM), jnp.float32)

    partials, cnts = _sc_segsum(x, batch2, rix2, ones_rows, zvec)
    p2 = partials.reshape(2 * N_GRAPHS, IN_DIM)
    c2 = cnts.reshape(2 * N_GRAPHS, IN_DIM)

    acc_tc = pl.pallas_call(
        _tc_body,
        grid=(NBLK,),
        in_specs=[
            pl.BlockSpec((R, IN_DIM), lambda i: (i, 0)),
            pl.BlockSpec((1, 1, R), lambda i: (i, 0, 0)),
            pl.BlockSpec((IN_DIM, HP), lambda i: (0, 0)),
        ],
        out_specs=pl.BlockSpec((N_GRAPHS, HP), lambda i: (0, 0)),
        out_shape=jax.ShapeDtypeStruct((N_GRAPHS, HP), jnp.float32),
        compiler_params=pltpu.CompilerParams(
            dimension_semantics=("arbitrary",),
        ),
    )(x, batch3, wt_pad)

    out = pl.pallas_call(
        _combine_body,
        in_specs=[
            pl.BlockSpec((N_GRAPHS, HP), lambda: (0, 0)),
            pl.BlockSpec((2 * N_GRAPHS, IN_DIM), lambda: (0, 0)),
            pl.BlockSpec((2 * N_GRAPHS, IN_DIM), lambda: (0, 0)),
            pl.BlockSpec((IN_DIM, HP), lambda: (0, 0)),
            pl.BlockSpec((1, NUM_CLASSES), lambda: (0, 0)),
        ],
        out_specs=pl.BlockSpec((N_GRAPHS, NUM_CLASSES), lambda: (0, 0)),
        out_shape=jax.ShapeDtypeStruct((N_GRAPHS, NUM_CLASSES), jnp.float32),
    )(acc_tc, p2, c2, wt_pad, b.reshape(1, NUM_CLASSES))
    return out
